# Initial kernel scaffold; baseline (speedup 1.0000x reference)
import jax
import jax.numpy as jnp
from jax.experimental import pallas as pl


def kernel(events, params):
    raise NotImplementedError("write your pallas kernel here")


# speedup vs baseline: 7.3686x; 7.3686x over previous
import functools

import numpy as np
import jax
import jax.numpy as jnp
from jax import lax
from jax.experimental import pallas as pl
from jax.experimental.pallas import tpu as pltpu
from jax.experimental.pallas import tpu_sc as plsc

N = 1024
M = 2048
KNN = 16
F = 512

_f32 = jnp.float32


def _dot(a, b):
    return jnp.dot(a, b, preferred_element_type=jnp.float32)


TW = 2 * F + 128


def _prep_body(e_ref, ev128_ref, w0_ref, b0_ref, w1_ref, b1_ref, w2_ref,
               b2_ref, kw_ref, kb_ref, wq_ref, wkv_ref,
               upf_ref, q_ref, xkv_ref):
    e = e_ref[...]
    f = jnp.maximum(_dot(e, w0_ref[...]) + b0_ref[...], 0.0)
    f = jnp.maximum(_dot(f, w1_ref[...]) + b1_ref[...], 0.0)
    f = _dot(f, w2_ref[...]) + b2_ref[...]
    c0 = _f32(2.0 / 3.0)
    c1 = _f32(1.0 / 3.0)
    upf = jnp.concatenate([f, f * c0 + f * c1], axis=0)
    upf_ref[...] = upf
    x = _dot(upf, kw_ref[...]) + kb_ref[...]
    q_ref[...] = _dot(x, wq_ref[...])
    xkv_ref[:, :2 * F] = _dot(x, wkv_ref[...])
    xkv_ref[:, 2 * F:] = ev128_ref[...]


def _prep(e, ev128, w0T, b0, w1T, b1, w2T, b2, kwT, kb, wqT, wkvT):
    full = lambda shape: pl.BlockSpec(shape, lambda: (0, 0))
    return pl.pallas_call(
        _prep_body,
        grid=(),
        in_specs=[full(a.shape) for a in
                  (e, ev128, w0T, b0, w1T, b1, w2T, b2, kwT, kb, wqT, wkvT)],
        out_specs=[full((M, F)), full((M, F)), full((M, TW))],
        out_shape=[jax.ShapeDtypeStruct((M, F), _f32),
                   jax.ShapeDtypeStruct((M, F), _f32),
                   jax.ShapeDtypeStruct((M, TW), _f32)],
    )(e, ev128, w0T, b0, w1T, b1, w2T, b2, kwT, kb, wqT, wkvT)


_KNN_TB = 256


def _knn_body(ev_ref, evt_ref, idx_ref):
    a = ev_ref[...]
    bt = evt_ref[...]
    d = jnp.zeros((_KNN_TB, M), _f32)
    for c in range(3):
        dc = a[:, c:c + 1] - bt[c:c + 1, :]
        d = d + dc * dc
    iota = lax.broadcasted_iota(jnp.int32, (_KNN_TB, M), 1)
    iota16 = lax.broadcasted_iota(jnp.int32, (_KNN_TB, KNN), 1)
    out = jnp.zeros((_KNN_TB, KNN), jnp.int32)
    inf = _f32(np.inf)
    for j in range(KNN):
        mn = jnp.min(d, axis=1, keepdims=True)
        am = jnp.min(jnp.where(d == mn, iota, M), axis=1, keepdims=True)
        out = jnp.where(iota16 == j, am, out)
        d = jnp.where(iota == am, inf, d)
    idx_ref[...] = out


def _knn(ev128, evt):
    grid = M // _KNN_TB
    return pl.pallas_call(
        _knn_body,
        grid=(grid,),
        in_specs=[pl.BlockSpec((_KNN_TB, 128), lambda i: (i, 0)),
                  pl.BlockSpec((8, M), lambda i: (0, 0))],
        out_specs=pl.BlockSpec((_KNN_TB, KNN), lambda i: (i, 0)),
        out_shape=jax.ShapeDtypeStruct((M, KNN), jnp.int32),
    )(ev128, evt)


_SC_CHUNK = 64


def _sc_gather(xkv, idxf):
    info = plsc.get_sparse_core_info()
    nw = info.num_cores * info.num_subcores
    b = idxf.shape[0]
    per_w = b // nw
    mesh = plsc.VectorSubcoreMesh(core_axis_name="c", subcore_axis_name="s")

    @functools.partial(
        pl.kernel,
        out_type=jax.ShapeDtypeStruct((b, TW), _f32),
        mesh=mesh,
        scratch_types=[pltpu.VMEM((_SC_CHUNK,), jnp.int32),
                       pltpu.VMEM((_SC_CHUNK, TW), _f32),
                       pltpu.SemaphoreType.DMA],
    )
    def gat(xkv_hbm, idx_hbm, okv_hbm, idx_v, rows_v, sem):
        wid = lax.axis_index("s") * info.num_cores + lax.axis_index("c")
        base = wid * per_w

        @pl.loop(0, per_w, step=_SC_CHUNK)
        def _(c):
            pltpu.sync_copy(idx_hbm.at[pl.ds(base + c, _SC_CHUNK)], idx_v)
            pltpu.async_copy(xkv_hbm.at[idx_v], rows_v, sem).wait()
            pltpu.sync_copy(rows_v, okv_hbm.at[pl.ds(base + c, _SC_CHUNK)])

    return gat(xkv, idxf)


_ATT_TB = 128


def _att_body(q_ref, evm_ref, upf_ref, kvg_ref,
              wm_ref, wk_ref, wd_ref, peb0_ref, pw1_ref, peb1_ref,
              a0_ref, ab0_ref, a1_ref, ab1_ref, ag_ref, agb_ref,
              ow_ref, ob_ref, out_ref):
    tb = _ATT_TB
    rows = tb * KNN
    evm = evm_ref[...]
    evk = kvg_ref[:, 2 * F:]
    pm = jnp.broadcast_to(evm[:, None, :], (tb, KNN, 128)).reshape(rows, 128)
    diff = pm - evk
    sq = jnp.sqrt(diff * diff + _f32(1e-12))
    h = _dot(pm, wm_ref[...]) + _dot(evk, wk_ref[...]) + _dot(sq, wd_ref[...])
    h = jnp.maximum(h + peb0_ref[...], 0.0)
    pe = _dot(h, pw1_ref[...]) + peb1_ref[...]
    kk = kvg_ref[:, :F]
    vv = kvg_ref[:, F:2 * F]
    q = q_ref[...]
    q_rep = jnp.broadcast_to(q[:, None, :], (tb, KNN, F)).reshape(rows, F)
    pre = q_rep - kk + pe
    t = jnp.maximum(_dot(pre, a0_ref[...]) + ab0_ref[...], 0.0)
    att = _dot(t, a1_ref[...]) + ab1_ref[...]
    att = att / _f32(np.sqrt(float(F)))
    att3 = att.reshape(tb, KNN, F)
    mx = jnp.max(att3, axis=1, keepdims=True)
    ex = jnp.exp(att3 - mx)
    sm = ex / jnp.sum(ex, axis=1, keepdims=True)
    pv3 = (vv + pe).reshape(tb, KNN, F)
    res = jnp.sum(sm * pv3, axis=1)
    res = _dot(res, ag_ref[...]) + agb_ref[...] + upf_ref[...]
    out_ref[...] = _dot(res, ow_ref[...]) + ob_ref[...]


def _att(q, ev128, upf, kvg, wmT, wkT, wdT, peb0, pw1T, peb1,
         a0T, ab0, a1T, ab1, agT, agb, owT, obp):
    grid = M // _ATT_TB
    rows = _ATT_TB * KNN
    cw = lambda a: pl.BlockSpec(a.shape, lambda i: (0, 0))
    return pl.pallas_call(
        _att_body,
        grid=(grid,),
        in_specs=[pl.BlockSpec((_ATT_TB, F), lambda i: (i, 0)),
                  pl.BlockSpec((_ATT_TB, 128), lambda i: (i, 0)),
                  pl.BlockSpec((_ATT_TB, F), lambda i: (i, 0)),
                  pl.BlockSpec((rows, TW), lambda i: (i, 0)),
                  cw(wmT), cw(wkT), cw(wdT), cw(peb0), cw(pw1T), cw(peb1),
                  cw(a0T), cw(ab0), cw(a1T), cw(ab1), cw(agT), cw(agb),
                  cw(owT), cw(obp)],
        out_specs=pl.BlockSpec((_ATT_TB, 128), lambda i: (i, 0)),
        out_shape=jax.ShapeDtypeStruct((M, 128), _f32),
    )(q, ev128, upf, kvg, wmT, wkT, wdT, peb0, pw1T, peb1,
      a0T, ab0, a1T, ab1, agT, agb, owT, obp)


def kernel(events, params):
    p = params
    e = events[0]
    c0 = _f32(2.0 / 3.0)
    c1 = _f32(1.0 / 3.0)
    ev = jnp.concatenate([e, e * c0 + e * c1], axis=0)
    ev128 = jnp.zeros((M, 128), _f32).at[:, :4].set(ev)
    evt = jnp.zeros((8, M), _f32).at[:3, :].set(ev[:, :3].T)

    row = lambda v: v.reshape(1, -1)
    w0T, w1T, w2T = p['emb_W0'].T, p['emb_W1'].T, p['emb_W2'].T
    kwT, wqT = p['ker_W'].T, p['wq'].T
    wkvT = jnp.concatenate([p['wk'].T, p['wv'].T], axis=1)
    pe0 = p['pe_W0']
    wmT = jnp.zeros((128, F), _f32).at[:4, :].set((pe0[:, 0:4] + pe0[:, 8:12]).T)
    wkT = jnp.zeros((128, F), _f32).at[:4, :].set((pe0[:, 4:8] - pe0[:, 8:12]).T)
    wdT = jnp.zeros((128, F), _f32).at[:4, :].set(pe0[:, 12:16].T)
    pw1T, a0T, a1T, agT = p['pe_W1'].T, p['att_W0'].T, p['att_W1'].T, p['agg_W'].T
    owT = jnp.zeros((F, 128), _f32).at[:, :3].set(p['out_W'].T)
    obp = jnp.zeros((1, 128), _f32).at[:, :3].set(p['out_b'][None, :])

    upf, q, xkv = _prep(e, ev128, w0T, row(p['emb_b0']), w1T, row(p['emb_b1']),
                        w2T, row(p['emb_b2']), kwT, row(p['ker_b']), wqT, wkvT)
    idx = _knn(ev128, evt)
    kvg = _sc_gather(xkv, idx.reshape(-1))
    out = _att(q, ev128, upf, kvg, wmT, wkT, wdT, row(p['pe_b0']),
               pw1T, row(p['pe_b1']), a0T, row(p['att_b0']), a1T,
               row(p['att_b1']), agT, row(p['agg_b']), owT, obp)
    return ev[None, :, :3], out[None, :, :3]


# speedup vs baseline: 10.9839x; 1.4906x over previous
import functools

import numpy as np
import jax
import jax.numpy as jnp
from jax import lax
from jax.experimental import pallas as pl
from jax.experimental.pallas import tpu as pltpu
from jax.experimental.pallas import tpu_sc as plsc

N = 1024
M = 2048
KNN = 16
F = 512
TW = F + 128

_f32 = jnp.float32


def _dot(a, w):
    return lax.dot_general(a, w, (((1,), (1,)), ((), ())),
                           preferred_element_type=jnp.float32)


_KNN_TB = 256


def _prep_body(e_ref, w0_ref, b0_ref, w1_ref, b1_ref, w2_ref, b2_ref,
               kw_ref, kb_ref, wq_ref, wkv_ref,
               upf_ref, q_ref, xkv_ref, ev128_ref, idx_ref):
    e = e_ref[...]
    f = jnp.maximum(_dot(e, w0_ref[...]) + b0_ref[...], 0.0)
    f = jnp.maximum(_dot(f, w1_ref[...]) + b1_ref[...], 0.0)
    f = _dot(f, w2_ref[...]) + b2_ref[...]
    c0 = _f32(2.0 / 3.0)
    c1 = _f32(1.0 / 3.0)
    upf = jnp.concatenate([f, f * c0 + f * c1], axis=0)
    upf_ref[...] = upf
    x = _dot(upf, kw_ref[...]) + kb_ref[...]
    q_ref[...] = _dot(x, wq_ref[...])
    kv = _dot(x, wkv_ref[...])
    ki = lax.bitcast_convert_type(kv[:, :F], jnp.int32)
    vi = lax.bitcast_convert_type(kv[:, F:], jnp.int32)
    ki = ki + 0x7FFF + ((ki >> 16) & 1)
    vi = vi + 0x7FFF + ((vi >> 16) & 1)
    packed = (ki & -65536) | ((vi >> 16) & 65535)
    xkv_ref[:, :F] = lax.bitcast_convert_type(packed, jnp.float32)

    ev4 = jnp.concatenate([e, e * c0 + e * c1], axis=0)
    ev128 = jnp.concatenate([ev4, jnp.zeros((M, 124), _f32)], axis=1)
    ev128_ref[...] = ev128
    xkv_ref[:, F:] = ev128

    ev8 = jnp.concatenate([ev4[:, :3], jnp.zeros((M, 5), _f32)], axis=1)
    bt = ev8.T
    iota = lax.broadcasted_iota(jnp.int32, (_KNN_TB, M), 1)
    iota16 = lax.broadcasted_iota(jnp.int32, (_KNN_TB, KNN), 1)
    inf = _f32(np.inf)

    def tile(t, _):
        a = ev128_ref[pl.ds(t * _KNN_TB, _KNN_TB), :]
        d = jnp.zeros((_KNN_TB, M), _f32)
        for c in range(3):
            dc = a[:, c:c + 1] - bt[c:c + 1, :]
            d = d + dc * dc
        out = jnp.zeros((_KNN_TB, KNN), jnp.int32)
        for j in range(KNN):
            am = jnp.argmin(d, axis=1).astype(jnp.int32)[:, None]
            out = jnp.where(iota16 == j, am, out)
            d = jnp.where(iota == am, inf, d)
        idx_ref[pl.ds(t * _KNN_TB, _KNN_TB), :] = out
        return 0

    lax.fori_loop(0, M // _KNN_TB, tile, 0)


def _prep(e, w0, b0, w1, b1, w2, b2, kw, kb, wq, wkv):
    full = lambda shape: pl.BlockSpec(shape, lambda: (0, 0))
    return pl.pallas_call(
        _prep_body,
        grid=(),
        in_specs=[full(a.shape) for a in
                  (e, w0, b0, w1, b1, w2, b2, kw, kb, wq, wkv)],
        out_specs=[full((M, F)), full((M, F)), full((M, TW)),
                   full((M, 128)), full((M, KNN))],
        out_shape=[jax.ShapeDtypeStruct((M, F), _f32),
                   jax.ShapeDtypeStruct((M, F), _f32),
                   jax.ShapeDtypeStruct((M, TW), _f32),
                   jax.ShapeDtypeStruct((M, 128), _f32),
                   jax.ShapeDtypeStruct((M, KNN), jnp.int32)],
    )(e, w0, b0, w1, b1, w2, b2, kw, kb, wq, wkv)


_SC_CHUNK = 64


def _sc_gather(xkv, idxf):
    info = plsc.get_sparse_core_info()
    nw = info.num_cores * info.num_subcores
    b = idxf.shape[0]
    per_w = b // nw
    mesh = plsc.VectorSubcoreMesh(core_axis_name="c", subcore_axis_name="s")

    nch = per_w // _SC_CHUNK

    @functools.partial(
        pl.kernel,
        out_type=jax.ShapeDtypeStruct((b, TW), _f32),
        mesh=mesh,
        scratch_types=[pltpu.VMEM((per_w,), jnp.int32),
                       pltpu.VMEM((_SC_CHUNK, TW), _f32),
                       pltpu.VMEM((_SC_CHUNK, TW), _f32),
                       pltpu.SemaphoreType.DMA,
                       pltpu.SemaphoreType.DMA,
                       pltpu.SemaphoreType.DMA],
    )
    def gat(xkv_hbm, idx_hbm, okv_hbm, idx_v, r0, r1, semg, s0, s1):
        wid = lax.axis_index("s") * info.num_cores + lax.axis_index("c")
        base = wid * per_w
        pltpu.sync_copy(idx_hbm.at[pl.ds(base, per_w)], idx_v)
        bufs, sems, pend = (r0, r1), (s0, s1), [None, None]
        for i in range(nch):
            bb = i & 1
            if pend[bb] is not None:
                pend[bb].wait()
            pltpu.async_copy(
                xkv_hbm.at[idx_v.at[pl.ds(i * _SC_CHUNK, _SC_CHUNK)]],
                bufs[bb], semg).wait()
            pend[bb] = pltpu.async_copy(
                bufs[bb], okv_hbm.at[pl.ds(base + i * _SC_CHUNK, _SC_CHUNK)],
                sems[bb])
        for bb in (0, 1):
            if pend[bb] is not None:
                pend[bb].wait()

    return gat(xkv, idxf)


_ATT_TB = 128


def _att_body(q_ref, evm_ref, upf_ref, kvg_ref,
              wm_ref, wk_ref, wd_ref, peb0_ref, pw1_ref, peb1_ref,
              a0_ref, ab0_ref, a1_ref, ab1_ref, ag_ref, agb_ref,
              ow_ref, ob_ref, out_ref):
    tb = _ATT_TB
    rows = tb * KNN
    evm = evm_ref[...]
    evk = kvg_ref[:, F:]
    pm = jnp.broadcast_to(evm[:, None, :], (tb, KNN, 128)).reshape(rows, 128)
    diff = pm - evk
    sq = jnp.sqrt(diff * diff + _f32(1e-12))
    ha = _dot(evm, wm_ref[...]) + peb0_ref[...]
    ha = jnp.broadcast_to(ha[:, None, :], (tb, KNN, F)).reshape(rows, F)
    h = ha + _dot(evk, wk_ref[...]) + _dot(sq, wd_ref[...])
    h = jnp.maximum(h, 0.0)
    pe = _dot(h, pw1_ref[...]) + peb1_ref[...]
    w = lax.bitcast_convert_type(kvg_ref[:, :F], jnp.int32)
    kk = lax.bitcast_convert_type(w & -65536, jnp.float32)
    vv = lax.bitcast_convert_type(w << 16, jnp.float32)
    q = q_ref[...]
    q_rep = jnp.broadcast_to(q[:, None, :], (tb, KNN, F)).reshape(rows, F)
    pre = q_rep - kk + pe
    t = jnp.maximum(_dot(pre, a0_ref[...]) + ab0_ref[...], 0.0)
    att = _dot(t, a1_ref[...]) + ab1_ref[...]
    ex = jnp.exp(att.reshape(tb, KNN, F))
    pv3 = (vv + pe).reshape(tb, KNN, F)
    res = jnp.sum(ex * pv3, axis=1) / jnp.sum(ex, axis=1)
    res = _dot(res, ag_ref[...]) + agb_ref[...] + upf_ref[...]
    out_ref[...] = _dot(res, ow_ref[...]) + ob_ref[...]


def _att(q, ev128, upf, kvg, wm, wk, wd, peb0, pw1, peb1,
         a0, ab0, a1, ab1, ag, agb, owp, obp):
    mt = q.shape[0]
    grid = mt // _ATT_TB
    rows = _ATT_TB * KNN
    cw = lambda a: pl.BlockSpec(a.shape, lambda i: (0, 0))
    return pl.pallas_call(
        _att_body,
        grid=(grid,),
        in_specs=[pl.BlockSpec((_ATT_TB, F), lambda i: (i, 0)),
                  pl.BlockSpec((_ATT_TB, 128), lambda i: (i, 0)),
                  pl.BlockSpec((_ATT_TB, F), lambda i: (i, 0)),
                  pl.BlockSpec((rows, TW), lambda i: (i, 0)),
                  cw(wm), cw(wk), cw(wd), cw(peb0), cw(pw1), cw(peb1),
                  cw(a0), cw(ab0), cw(a1), cw(ab1), cw(ag), cw(agb),
                  cw(owp), cw(obp)],
        out_specs=pl.BlockSpec((_ATT_TB, 128), lambda i: (i, 0)),
        out_shape=jax.ShapeDtypeStruct((mt, 128), _f32),
    )(q, ev128, upf, kvg, wm, wk, wd, peb0, pw1, peb1,
      a0, ab0, a1, ab1, ag, agb, owp, obp)


_HALVES = 2


def kernel(events, params):
    p = params
    e = events[0]

    row = lambda v: v.reshape(1, -1)
    wkv = jnp.concatenate([p['wk'], p['wv']], axis=0)
    pe0 = p['pe_W0']
    wm = jnp.zeros((F, 128), _f32).at[:, :4].set(pe0[:, 0:4] + pe0[:, 8:12])
    wk = jnp.zeros((F, 128), _f32).at[:, :4].set(pe0[:, 4:8] - pe0[:, 8:12])
    wd = jnp.zeros((F, 128), _f32).at[:, :4].set(pe0[:, 12:16])
    inv_s = _f32(1.0 / np.sqrt(float(F)))
    a1 = p['att_W1'] * inv_s
    ab1 = p['att_b1'] * inv_s
    owp = jnp.zeros((128, F), _f32).at[:3, :].set(p['out_W'])
    obp = jnp.zeros((1, 128), _f32).at[:, :3].set(p['out_b'][None, :])

    upf, q, xkv, ev128, idx = _prep(
        e, p['emb_W0'], row(p['emb_b0']), p['emb_W1'], row(p['emb_b1']),
        p['emb_W2'], row(p['emb_b2']), p['ker_W'], row(p['ker_b']),
        p['wq'], wkv)

    mh = M // _HALVES
    outs = []
    for hh in range(_HALVES):
        s = slice(hh * mh, (hh + 1) * mh)
        kvg = _sc_gather(xkv, idx[s].reshape(-1))
        outs.append(_att(q[s], ev128[s], upf[s], kvg, wm, wk, wd,
                         row(p['pe_b0']), p['pe_W1'], row(p['pe_b1']),
                         p['att_W0'], row(p['att_b0']), a1, row(ab1),
                         p['agg_W'], row(p['agg_b']), owp, obp))
    out = jnp.concatenate(outs, axis=0)
    return ev128[None, :, :3], out[None, :, :3]


# speedup vs baseline: 10.9845x; 1.0001x over previous
import functools

import numpy as np
import jax
import jax.numpy as jnp
from jax import lax
from jax.experimental import pallas as pl
from jax.experimental.pallas import tpu as pltpu
from jax.experimental.pallas import tpu_sc as plsc

N = 1024
M = 2048
KNN = 16
F = 512
TW = F + 128

_f32 = jnp.float32


def _dot(a, w):
    return lax.dot_general(a, w, (((1,), (1,)), ((), ())),
                           preferred_element_type=jnp.float32)


_KNN_TB = 256


def _prep_body(e_ref, w0_ref, b0_ref, w1_ref, b1_ref, w2_ref, b2_ref,
               kw_ref, kb_ref, wq_ref, wkv_ref,
               upf_ref, q_ref, xkv_ref, ev128_ref, idx_ref):
    e = e_ref[...]
    f = jnp.maximum(_dot(e, w0_ref[...]) + b0_ref[...], 0.0)
    f = jnp.maximum(_dot(f, w1_ref[...]) + b1_ref[...], 0.0)
    f = _dot(f, w2_ref[...]) + b2_ref[...]
    c0 = _f32(2.0 / 3.0)
    c1 = _f32(1.0 / 3.0)
    upf = jnp.concatenate([f, f * c0 + f * c1], axis=0)
    upf_ref[...] = upf
    x = _dot(upf, kw_ref[...]) + kb_ref[...]
    q_ref[...] = _dot(x, wq_ref[...])
    kv = _dot(x, wkv_ref[...])
    ki = lax.bitcast_convert_type(kv[:, :F], jnp.int32)
    vi = lax.bitcast_convert_type(kv[:, F:], jnp.int32)
    ki = ki + 0x7FFF + ((ki >> 16) & 1)
    vi = vi + 0x7FFF + ((vi >> 16) & 1)
    packed = (ki & -65536) | ((vi >> 16) & 65535)
    xkv_ref[:, :F] = lax.bitcast_convert_type(packed, jnp.float32)

    ev4 = jnp.concatenate([e, e * c0 + e * c1], axis=0)
    ev128 = jnp.concatenate([ev4, jnp.zeros((M, 124), _f32)], axis=1)
    ev128_ref[...] = ev128
    xkv_ref[:, F:] = ev128

    ev8 = jnp.concatenate([ev4[:, :3], jnp.zeros((M, 5), _f32)], axis=1)
    bt = ev8.T
    iota = lax.broadcasted_iota(jnp.int32, (_KNN_TB, M), 1)
    iota16 = lax.broadcasted_iota(jnp.int32, (_KNN_TB, KNN), 1)
    inf = _f32(np.inf)

    def tile(t, _):
        a = ev128_ref[pl.ds(t * _KNN_TB, _KNN_TB), :]
        d = jnp.zeros((_KNN_TB, M), _f32)
        for c in range(3):
            dc = a[:, c:c + 1] - bt[c:c + 1, :]
            d = d + dc * dc
        out = jnp.zeros((_KNN_TB, KNN), jnp.int32)
        for j in range(KNN):
            am = jnp.argmin(d, axis=1).astype(jnp.int32)[:, None]
            out = jnp.where(iota16 == j, am, out)
            d = jnp.where(iota == am, inf, d)
        idx_ref[pl.ds(t * _KNN_TB, _KNN_TB), :] = out
        return 0

    lax.fori_loop(0, M // _KNN_TB, tile, 0)


def _prep(e, w0, b0, w1, b1, w2, b2, kw, kb, wq, wkv):
    full = lambda shape: pl.BlockSpec(shape, lambda: (0, 0))
    return pl.pallas_call(
        _prep_body,
        grid=(),
        in_specs=[full(a.shape) for a in
                  (e, w0, b0, w1, b1, w2, b2, kw, kb, wq, wkv)],
        out_specs=[full((M, F)), full((M, F)), full((M, TW)),
                   full((M, 128)), full((M, KNN))],
        out_shape=[jax.ShapeDtypeStruct((M, F), _f32),
                   jax.ShapeDtypeStruct((M, F), _f32),
                   jax.ShapeDtypeStruct((M, TW), _f32),
                   jax.ShapeDtypeStruct((M, 128), _f32),
                   jax.ShapeDtypeStruct((M, KNN), jnp.int32)],
    )(e, w0, b0, w1, b1, w2, b2, kw, kb, wq, wkv)


_SC_CHUNK = 64


def _sc_gather(xkv, idxf):
    info = plsc.get_sparse_core_info()
    nw = info.num_cores * info.num_subcores
    b = idxf.shape[0]
    per_w = b // nw
    mesh = plsc.VectorSubcoreMesh(core_axis_name="c", subcore_axis_name="s")

    nch = per_w // _SC_CHUNK

    @functools.partial(
        pl.kernel,
        out_type=jax.ShapeDtypeStruct((b, TW), _f32),
        mesh=mesh,
        scratch_types=[pltpu.VMEM((per_w,), jnp.int32),
                       pltpu.VMEM((_SC_CHUNK, TW), _f32),
                       pltpu.VMEM((_SC_CHUNK, TW), _f32),
                       pltpu.SemaphoreType.DMA,
                       pltpu.SemaphoreType.DMA,
                       pltpu.SemaphoreType.DMA],
    )
    def gat(xkv_hbm, idx_hbm, okv_hbm, idx_v, r0, r1, semg, s0, s1):
        wid = lax.axis_index("s") * info.num_cores + lax.axis_index("c")
        base = wid * per_w
        pltpu.sync_copy(idx_hbm.at[pl.ds(base, per_w)], idx_v)
        bufs, sems, pend = (r0, r1), (s0, s1), [None, None]
        for i in range(nch):
            bb = i & 1
            if pend[bb] is not None:
                pend[bb].wait()
            pltpu.async_copy(
                xkv_hbm.at[idx_v.at[pl.ds(i * _SC_CHUNK, _SC_CHUNK)]],
                bufs[bb], semg).wait()
            pend[bb] = pltpu.async_copy(
                bufs[bb], okv_hbm.at[pl.ds(base + i * _SC_CHUNK, _SC_CHUNK)],
                sems[bb])
        for bb in (0, 1):
            if pend[bb] is not None:
                pend[bb].wait()

    return gat(xkv, idxf)


_ATT_TB = 128


def _att_body(q_ref, evm_ref, upf_ref, kvg_ref,
              wm_ref, wk_ref, wd_ref, peb0_ref, pw1_ref, peb1_ref,
              a0_ref, ab0_ref, a1_ref, ab1_ref, ag_ref, agb_ref,
              ow_ref, ob_ref, out_ref):
    tb = _ATT_TB
    rows = tb * KNN
    evm = evm_ref[...]
    evk = kvg_ref[:, F:]
    pm = jnp.broadcast_to(evm[:, None, :], (tb, KNN, 128)).reshape(rows, 128)
    diff = pm - evk
    sq = jnp.sqrt(diff * diff + _f32(1e-12))
    ha = _dot(evm, wm_ref[...]) + peb0_ref[...]
    ha = jnp.broadcast_to(ha[:, None, :], (tb, KNN, F)).reshape(rows, F)
    h = ha + _dot(evk, wk_ref[...]) + _dot(sq, wd_ref[...])
    h = jnp.maximum(h, 0.0)
    pe = _dot(h, pw1_ref[...]) + peb1_ref[...]
    w = lax.bitcast_convert_type(kvg_ref[:, :F], jnp.int32)
    kk = lax.bitcast_convert_type(w & -65536, jnp.float32)
    vv = lax.bitcast_convert_type(w << 16, jnp.float32)
    q = q_ref[...]
    q_rep = jnp.broadcast_to(q[:, None, :], (tb, KNN, F)).reshape(rows, F)
    pre = q_rep - kk + pe
    t = jnp.maximum(_dot(pre, a0_ref[...]) + ab0_ref[...], 0.0)
    att = _dot(t, a1_ref[...]) + ab1_ref[...]
    ex = jnp.exp(att.reshape(tb, KNN, F))
    pv3 = (vv + pe).reshape(tb, KNN, F)
    res = jnp.sum(ex * pv3, axis=1) / jnp.sum(ex, axis=1)
    res = _dot(res, ag_ref[...]) + agb_ref[...] + upf_ref[...]
    out_ref[...] = _dot(res, ow_ref[...]) + ob_ref[...]


def _att(q, ev128, upf, kvg, wm, wk, wd, peb0, pw1, peb1,
         a0, ab0, a1, ab1, ag, agb, owp, obp):
    mt = q.shape[0]
    grid = mt // _ATT_TB
    rows = _ATT_TB * KNN
    cw = lambda a: pl.BlockSpec(a.shape, lambda i: (0, 0))
    return pl.pallas_call(
        _att_body,
        grid=(grid,),
        in_specs=[pl.BlockSpec((_ATT_TB, F), lambda i: (i, 0)),
                  pl.BlockSpec((_ATT_TB, 128), lambda i: (i, 0)),
                  pl.BlockSpec((_ATT_TB, F), lambda i: (i, 0)),
                  pl.BlockSpec((rows, TW), lambda i: (i, 0)),
                  cw(wm), cw(wk), cw(wd), cw(peb0), cw(pw1), cw(peb1),
                  cw(a0), cw(ab0), cw(a1), cw(ab1), cw(ag), cw(agb),
                  cw(owp), cw(obp)],
        out_specs=pl.BlockSpec((_ATT_TB, 128), lambda i: (i, 0)),
        out_shape=jax.ShapeDtypeStruct((mt, 128), _f32),
    )(q, ev128, upf, kvg, wm, wk, wd, peb0, pw1, peb1,
      a0, ab0, a1, ab1, ag, agb, owp, obp)


_HALVES = 2


def kernel(events, params):
    p = params
    e = events[0]

    row = lambda v: v.reshape(1, -1)
    wkv = jnp.concatenate([p['wk'], p['wv']], axis=0)
    pe0 = p['pe_W0']
    wm = jnp.zeros((F, 128), _f32).at[:, :4].set(pe0[:, 0:4] + pe0[:, 8:12])
    wk = jnp.zeros((F, 128), _f32).at[:, :4].set(pe0[:, 4:8] - pe0[:, 8:12])
    wd = jnp.zeros((F, 128), _f32).at[:, :4].set(pe0[:, 12:16])
    inv_s = _f32(1.0 / np.sqrt(float(F)))
    a1 = p['att_W1'] * inv_s
    ab1 = p['att_b1'] * inv_s
    owp = jnp.zeros((128, F), _f32).at[:3, :].set(p['out_W'])
    obp = jnp.zeros((1, 128), _f32).at[:, :3].set(p['out_b'][None, :])

    upf, q, xkv, ev128, idx = _prep(
        e, p['emb_W0'], row(p['emb_b0']), p['emb_W1'], row(p['emb_b1']),
        p['emb_W2'], row(p['emb_b2']), p['ker_W'], row(p['ker_b']),
        p['wq'], wkv)

    mh = M // _HALVES
    slices = [slice(hh * mh, (hh + 1) * mh) for hh in range(_HALVES)]
    kvgs = [_sc_gather(xkv, idx[s].reshape(-1)) for s in slices]
    outs = [_att(q[s], ev128[s], upf[s], kvg, wm, wk, wd,
                 row(p['pe_b0']), p['pe_W1'], row(p['pe_b1']),
                 p['att_W0'], row(p['att_b0']), a1, row(ab1),
                 p['agg_W'], row(p['agg_b']), owp, obp)
            for s, kvg in zip(slices, kvgs)]
    out = jnp.concatenate(outs, axis=0)
    return ev128[None, :, :3], out[None, :, :3]
